# baseline (speedup 1.0000x reference)
import functools

import jax
import jax.numpy as jnp
from jax import lax
from jax.experimental import pallas as pl
from jax.experimental.pallas import tpu as pltpu
from jax.experimental.pallas import tpu_sc as plsc

VOCAB = 1000
OUT_DIM = 2
BATCH = 16384

_NC = 1
_NS = 16
_LANES = 16
_NW = _NC * _NS
_B_PER_W = BATCH // _NW
_GROUPS = _B_PER_W // _LANES
_HALF = _GROUPS // 2


def _gather_body(table_hbm, idx_hbm, out_hbm, table_v, idx_v, out_v,
                 sem_t, sem_i, sem_o):
    wid = lax.axis_index("s") * _NC + lax.axis_index("c")
    base = wid * _B_PER_W

    cp_t = pltpu.async_copy(table_hbm, table_v, sem_t)
    cp_i = pltpu.async_copy(idx_hbm.at[pl.ds(base, _B_PER_W)], idx_v, sem_i)
    cp_t.wait()
    cp_i.wait()

    lane = lax.iota(jnp.int32, _LANES)

    def gather_group(g):
        inds = idx_v[pl.ds(g * _LANES, _LANES)]
        flat0 = inds * OUT_DIM
        col0 = plsc.load_gather(table_v, [flat0])
        col1 = plsc.load_gather(table_v, [flat0 + 1])
        loc = (g * _LANES) * OUT_DIM + lane * OUT_DIM
        plsc.store_scatter(out_v, [loc], col0)
        plsc.store_scatter(out_v, [loc + 1], col1)

    half_elems = _HALF * _LANES * OUT_DIM

    for g in range(_HALF):
        gather_group(g)

    cp_o = pltpu.async_copy(
        out_v.at[pl.ds(0, half_elems)],
        out_hbm.at[pl.ds(base * OUT_DIM, half_elems)],
        sem_o,
    )

    for g in range(_HALF, _GROUPS):
        gather_group(g)

    pltpu.sync_copy(
        out_v.at[pl.ds(half_elems, half_elems)],
        out_hbm.at[pl.ds(base * OUT_DIM + half_elems, half_elems)],
    )
    cp_o.wait()


@jax.jit
def _run(table_flat, idx):
    mesh = plsc.VectorSubcoreMesh(core_axis_name="c", subcore_axis_name="s", num_cores=1)
    k = functools.partial(
        pl.kernel,
        mesh=mesh,
        out_type=jax.ShapeDtypeStruct((BATCH * OUT_DIM,), jnp.float32),
        scratch_types=[
            pltpu.VMEM((VOCAB * OUT_DIM,), jnp.float32),
            pltpu.VMEM((_B_PER_W,), jnp.int32),
            pltpu.VMEM((_B_PER_W * OUT_DIM,), jnp.float32),
            pltpu.SemaphoreType.DMA,
            pltpu.SemaphoreType.DMA,
            pltpu.SemaphoreType.DMA,
        ],
        compiler_params=pltpu.CompilerParams(needs_layout_passes=False),
    )(_gather_body)
    return k(table_flat, idx)


def kernel(B, u):
    table_flat = B.reshape(-1).astype(jnp.float32)
    idx = u.reshape(-1).astype(jnp.int32)
    out_flat = _run(table_flat, idx)
    return out_flat.reshape(BATCH, OUT_DIM)


# speedup vs baseline: 1.0422x; 1.0422x over previous
import functools

import jax
import jax.numpy as jnp
from jax import lax
from jax.experimental import pallas as pl
from jax.experimental.pallas import tpu as pltpu
from jax.experimental.pallas import tpu_sc as plsc

VOCAB = 1000
OUT_DIM = 2
BATCH = 16384

_NC = 1
_NS = 16
_LANES = 16
_NW = _NC * _NS
_B_PER_W = BATCH // _NW
_GROUPS = _B_PER_W // _LANES
_HALF = _GROUPS // 2


def _gather_body(table_hbm, idx_hbm, out_hbm, table_v, idx_v, out_v,
                 sem_t, sem_i, sem_o):
    wid = lax.axis_index("s") * _NC + lax.axis_index("c")
    base = wid * _B_PER_W

    cp_t = pltpu.async_copy(table_hbm, table_v, sem_t)
    cp_i = pltpu.async_copy(idx_hbm.at[pl.ds(base, _B_PER_W)], idx_v, sem_i)
    cp_t.wait()
    cp_i.wait()

    lane = lax.iota(jnp.int32, _LANES)

    def gather_group(g):
        inds = idx_v[pl.ds(g * _LANES, _LANES)]
        flat0 = inds * OUT_DIM
        col0 = plsc.load_gather(table_v, [flat0])
        col1 = plsc.load_gather(table_v, [flat0 + 1])
        loc = (g * _LANES) * OUT_DIM + lane * OUT_DIM
        plsc.store_scatter(out_v, [loc], col0)
        plsc.store_scatter(out_v, [loc + 1], col1)

    @plsc.parallel_loop(0, _GROUPS, unroll=8)
    def _(g):
        gather_group(g)

    pltpu.sync_copy(out_v, out_hbm.at[pl.ds(base * OUT_DIM, _B_PER_W * OUT_DIM)])


@jax.jit
def _run(table_flat, idx):
    mesh = plsc.VectorSubcoreMesh(core_axis_name="c", subcore_axis_name="s", num_cores=1)
    k = functools.partial(
        pl.kernel,
        mesh=mesh,
        out_type=jax.ShapeDtypeStruct((BATCH * OUT_DIM,), jnp.float32),
        scratch_types=[
            pltpu.VMEM((VOCAB * OUT_DIM,), jnp.float32),
            pltpu.VMEM((_B_PER_W,), jnp.int32),
            pltpu.VMEM((_B_PER_W * OUT_DIM,), jnp.float32),
            pltpu.SemaphoreType.DMA,
            pltpu.SemaphoreType.DMA,
            pltpu.SemaphoreType.DMA,
        ],
        compiler_params=pltpu.CompilerParams(needs_layout_passes=False),
    )(_gather_body)
    return k(table_flat, idx)


def kernel(B, u):
    table_flat = B.reshape(-1).astype(jnp.float32)
    idx = u.reshape(-1).astype(jnp.int32)
    out_flat = _run(table_flat, idx)
    return out_flat.reshape(BATCH, OUT_DIM)


# speedup vs baseline: 1.0456x; 1.0032x over previous
import functools

import jax
import jax.numpy as jnp
from jax import lax
from jax.experimental import pallas as pl
from jax.experimental.pallas import tpu as pltpu
from jax.experimental.pallas import tpu_sc as plsc

VOCAB = 1000
OUT_DIM = 2
BATCH = 16384

_NC = 1
_NS = 16
_LANES = 16
_NW = _NC * _NS
_B_PER_W = BATCH // _NW
_GROUPS = _B_PER_W // _LANES
_HALF = _GROUPS // 2


def _gather_body(table_hbm, idx_hbm, out_hbm, table_v, idx_v, out_v,
                 sem_t, sem_i, sem_o):
    wid = lax.axis_index("s") * _NC + lax.axis_index("c")
    base = wid * _B_PER_W

    cp_t = pltpu.async_copy(table_hbm, table_v, sem_t)
    cp_i = pltpu.async_copy(idx_hbm.at[pl.ds(base, _B_PER_W)], idx_v, sem_i)
    cp_t.wait()
    cp_i.wait()

    lane = lax.iota(jnp.int32, _LANES)

    def gather_group(g):
        inds = idx_v[pl.ds(g * _LANES, _LANES)]
        flat0 = inds * OUT_DIM
        col0 = plsc.load_gather(table_v, [flat0])
        col1 = plsc.load_gather(table_v, [flat0 + 1])
        loc = (g * _LANES) * OUT_DIM + lane * OUT_DIM
        plsc.store_scatter(out_v, [loc], col0)
        plsc.store_scatter(out_v, [loc + 1], col1)

    @plsc.parallel_loop(0, _GROUPS, unroll=4)
    def _(g):
        gather_group(g)

    pltpu.sync_copy(out_v, out_hbm.at[pl.ds(base * OUT_DIM, _B_PER_W * OUT_DIM)])


@jax.jit
def _run(table_flat, idx):
    mesh = plsc.VectorSubcoreMesh(core_axis_name="c", subcore_axis_name="s", num_cores=1)
    k = functools.partial(
        pl.kernel,
        mesh=mesh,
        out_type=jax.ShapeDtypeStruct((BATCH * OUT_DIM,), jnp.float32),
        scratch_types=[
            pltpu.VMEM((VOCAB * OUT_DIM,), jnp.float32),
            pltpu.VMEM((_B_PER_W,), jnp.int32),
            pltpu.VMEM((_B_PER_W * OUT_DIM,), jnp.float32),
            pltpu.SemaphoreType.DMA,
            pltpu.SemaphoreType.DMA,
            pltpu.SemaphoreType.DMA,
        ],
        compiler_params=pltpu.CompilerParams(needs_layout_passes=False),
    )(_gather_body)
    return k(table_flat, idx)


def kernel(B, u):
    table_flat = B.reshape(-1).astype(jnp.float32)
    idx = u.reshape(-1).astype(jnp.int32)
    out_flat = _run(table_flat, idx)
    return out_flat.reshape(BATCH, OUT_DIM)


# speedup vs baseline: 1.0459x; 1.0003x over previous
import functools

import jax
import jax.numpy as jnp
from jax import lax
from jax.experimental import pallas as pl
from jax.experimental.pallas import tpu as pltpu
from jax.experimental.pallas import tpu_sc as plsc

VOCAB = 1000
OUT_DIM = 2
BATCH = 16384

_NC = 1
_NS = 16
_LANES = 16
_NW = _NC * _NS
_B_PER_W = BATCH // _NW
_GROUPS = _B_PER_W // _LANES
_HALF = _GROUPS // 2


def _gather_body(table_hbm, idx_hbm, out_hbm, table_v, idx_v, out_v,
                 sem_t, sem_i, sem_o):
    wid = lax.axis_index("s") * _NC + lax.axis_index("c")
    base = wid * _B_PER_W

    cp_t = pltpu.async_copy(table_hbm, table_v, sem_t)
    cp_i = pltpu.async_copy(idx_hbm.at[pl.ds(base, _B_PER_W)], idx_v, sem_i)
    cp_t.wait()
    cp_i.wait()

    lane = lax.iota(jnp.int32, _LANES)

    def gather_group(g):
        inds = idx_v[pl.ds(g * _LANES, _LANES)]
        flat0 = inds * OUT_DIM
        col0 = plsc.load_gather(table_v, [flat0])
        col1 = plsc.load_gather(table_v, [flat0 + 1])
        loc = (g * _LANES) * OUT_DIM + lane * OUT_DIM
        plsc.store_scatter(out_v, [loc], col0)
        plsc.store_scatter(out_v, [loc + 1], col1)

    @plsc.parallel_loop(0, _GROUPS, unroll=2)
    def _(g):
        gather_group(g)

    pltpu.sync_copy(out_v, out_hbm.at[pl.ds(base * OUT_DIM, _B_PER_W * OUT_DIM)])


@jax.jit
def _run(table_flat, idx):
    mesh = plsc.VectorSubcoreMesh(core_axis_name="c", subcore_axis_name="s", num_cores=1)
    k = functools.partial(
        pl.kernel,
        mesh=mesh,
        out_type=jax.ShapeDtypeStruct((BATCH * OUT_DIM,), jnp.float32),
        scratch_types=[
            pltpu.VMEM((VOCAB * OUT_DIM,), jnp.float32),
            pltpu.VMEM((_B_PER_W,), jnp.int32),
            pltpu.VMEM((_B_PER_W * OUT_DIM,), jnp.float32),
            pltpu.SemaphoreType.DMA,
            pltpu.SemaphoreType.DMA,
            pltpu.SemaphoreType.DMA,
        ],
        compiler_params=pltpu.CompilerParams(needs_layout_passes=False),
    )(_gather_body)
    return k(table_flat, idx)


def kernel(B, u):
    table_flat = B.reshape(-1).astype(jnp.float32)
    idx = u.reshape(-1).astype(jnp.int32)
    out_flat = _run(table_flat, idx)
    return out_flat.reshape(BATCH, OUT_DIM)
